# Initial kernel scaffold; baseline (speedup 1.0000x reference)
import jax
import jax.numpy as jnp
from jax.experimental import pallas as pl


def kernel(x, edge_index, W, A):
    raise NotImplementedError("write your pallas kernel here")


# speedup vs baseline: 9.2269x; 9.2269x over previous
import functools

import jax
import jax.numpy as jnp
from jax import lax
from jax.experimental import pallas as pl
from jax.experimental.pallas import tpu as pltpu
from jax.experimental.pallas import tpu_sc as plsc

NF = 128
HH = 8
OF = 32
FT = HH * OF
NC = 2
NS = 16
NW = NC * NS
LL = 16

_NEG = -3.4e38


def _mesh():
    return plsc.VectorSubcoreMesh(core_axis_name="c", subcore_axis_name="s",
                                  num_cores=NC, num_subcores=NS)


def _wid():
    return lax.axis_index("s") * NC + lax.axis_index("c")


def _splat(i):
    return jnp.full((LL,), i, jnp.int32)


def _tc_body(x_ref, w_ref, as_ref, at_ref, nt_ref, als_ref, alt_ref):
    ntv = jnp.dot(x_ref[...], w_ref[...], preferred_element_type=jnp.float32)
    nt_ref[...] = ntv
    als_ref[...] = jnp.dot(ntv, as_ref[...], preferred_element_type=jnp.float32)
    alt_ref[...] = jnp.dot(ntv, at_ref[...], preferred_element_type=jnp.float32)


def _tc_project(x, w, a_s, a_t):
    n = x.shape[0]
    blk = 2000
    grid = (n // blk,)
    return pl.pallas_call(
        _tc_body,
        grid=grid,
        in_specs=[
            pl.BlockSpec((blk, NF), lambda i: (i, 0)),
            pl.BlockSpec((NF, FT), lambda i: (0, 0)),
            pl.BlockSpec((FT, LL), lambda i: (0, 0)),
            pl.BlockSpec((FT, LL), lambda i: (0, 0)),
        ],
        out_specs=[
            pl.BlockSpec((blk, FT), lambda i: (i, 0)),
            pl.BlockSpec((blk, LL), lambda i: (i, 0)),
            pl.BlockSpec((blk, LL), lambda i: (i, 0)),
        ],
        out_shape=[
            jax.ShapeDtypeStruct((n, FT), jnp.float32),
            jax.ShapeDtypeStruct((n, LL), jnp.float32),
            jax.ShapeDtypeStruct((n, LL), jnp.float32),
        ],
    )(x, w, a_s, a_t)


def _p1_body(src_h, tgt_h, als_h, alt_h, att_h, max_h,
             sidx, tidx, g1, g2, g3, sem):
    e = src_h.shape[0]
    epw = e // NW
    bsz = 400
    nb = epw // bsz
    wid = _wid()
    base0 = wid * epw
    lanes = lax.iota(jnp.int32, LL)
    lo_mask = lanes < HH

    def batch(b, mx):
        base = base0 + b * bsz
        pltpu.sync_copy(src_h.at[pl.ds(base, bsz)], sidx)
        pltpu.sync_copy(tgt_h.at[pl.ds(base, bsz)], tidx)
        pltpu.async_copy(als_h.at[sidx], g1, sem).wait()
        pltpu.async_copy(alt_h.at[tidx], g2, sem).wait()

        def row(i, mx):
            a = plsc.load_gather(g1, [_splat(i), lanes])
            bt = plsc.load_gather(g2, [_splat(i), lanes])
            v = a + bt
            g3[pl.ds(i * LL, LL)] = v
            sel = jnp.where(lo_mask, v, _NEG)
            return jnp.maximum(mx, sel)

        mx = lax.fori_loop(0, bsz, row, mx)
        pltpu.sync_copy(g3, att_h.at[pl.ds(base * LL, bsz * LL)])
        return mx

    mx = lax.fori_loop(0, nb, batch, jnp.full((LL,), _NEG, jnp.float32))
    g3[pl.ds(0, LL)] = mx
    pltpu.sync_copy(g3.at[pl.ds(0, LL)], max_h.at[pl.ds(wid * LL, LL)])


def _sc_attmax(src, tgt, als, alt):
    e = src.shape[0]
    bsz = 400
    kfn = pl.kernel(
        _p1_body,
        out_type=[
            jax.ShapeDtypeStruct((e * LL,), jnp.float32),
            jax.ShapeDtypeStruct((NW * LL,), jnp.float32),
        ],
        mesh=_mesh(),
        compiler_params=pltpu.CompilerParams(
            needs_layout_passes=False, use_tc_tiling_on_sc=False),
        scratch_types=[
            pltpu.VMEM((bsz,), jnp.int32),
            pltpu.VMEM((bsz,), jnp.int32),
            pltpu.VMEM((bsz, LL), jnp.float32),
            pltpu.VMEM((bsz, LL), jnp.float32),
            pltpu.VMEM((bsz * LL,), jnp.float32),
            pltpu.SemaphoreType.DMA,
        ],
    )
    return kfn(src, tgt, als, alt)


def _p2_body(tgt_h, att_h, max_h, zden_h, ex_h, den_h,
             tl, ab, eb, mb, den_sh, sem):
    e = tgt_h.shape[0]
    npad = den_sh.shape[0]
    drs = npad // NS
    epw = e // NW
    bsz = 400
    nb = epw // bsz
    wid = _wid()
    cid = lax.axis_index("c")
    sid = lax.axis_index("s")
    base0 = wid * epw
    lanes = lax.iota(jnp.int32, LL)
    lo_mask = lanes < HH

    pltpu.sync_copy(max_h, mb)

    def mred(i, mx):
        return jnp.maximum(mx, mb[pl.ds(i * LL, LL)])

    m16 = lax.fori_loop(0, NW, mred, jnp.full((LL,), _NEG, jnp.float32))
    m = jnp.max(m16)
    msp = jnp.full((LL,), m, jnp.float32)

    pltpu.sync_copy(zden_h, den_sh.at[pl.ds(sid * drs, drs)])
    plsc.subcore_barrier()

    def batch(b, carry):
        base = base0 + b * bsz
        pltpu.sync_copy(tgt_h.at[pl.ds(base, bsz)], tl)
        pltpu.sync_copy(att_h.at[pl.ds(base * LL, bsz * LL)], ab)

        def row(i, carry):
            v = ab[pl.ds(i * LL, LL)]
            d = v - msp
            w = jnp.where(d >= 0.0, d, d * jnp.float32(0.01))
            ev = jnp.exp(w)
            ev = jnp.where(lo_mask, ev, jnp.float32(0.0))
            plsc.store_scatter(eb, [_splat(i), lanes], ev)
            return carry

        carry = lax.fori_loop(0, bsz, row, carry)
        pltpu.sync_copy(eb, ex_h.at[pl.ds(base, bsz)])
        pltpu.sync_copy(eb, den_sh.at[tl], add=True)
        return carry

    lax.fori_loop(0, nb, batch, jnp.int32(0))
    plsc.subcore_barrier()
    pltpu.sync_copy(den_sh.at[pl.ds(sid * drs, drs)],
                    den_h.at[pl.ds(cid * npad + sid * drs, drs)])


def _sc_softmax_denom(tgt, att, mx, zden, npad):
    e = tgt.shape[0]
    bsz = 400
    kfn = pl.kernel(
        _p2_body,
        out_type=[
            jax.ShapeDtypeStruct((e, LL), jnp.float32),
            jax.ShapeDtypeStruct((NC * npad, LL), jnp.float32),
        ],
        mesh=_mesh(),
        compiler_params=pltpu.CompilerParams(
            needs_layout_passes=False, use_tc_tiling_on_sc=False),
        scratch_types=[
            pltpu.VMEM((bsz,), jnp.int32),
            pltpu.VMEM((bsz * LL,), jnp.float32),
            pltpu.VMEM((bsz, LL), jnp.float32),
            pltpu.VMEM((NW * LL,), jnp.float32),
            pltpu.VMEM_SHARED((npad, LL), jnp.float32),
            pltpu.SemaphoreType.DMA,
        ],
    )
    return kfn(tgt, att, mx, zden)


def _p3_body(src_h, tgt_h, ex_h, nt_h, den_h, zacc_h, out_h,
             acc, tgtb, srcb, slist, tlist, elist, ntb, exb,
             d0b, d1b, sem):
    e = src_h.shape[0]
    npad = out_h.shape[0]
    rpw = npad // NW
    ch = 2000
    nch = e // ch
    gsz = 64
    wid = _wid()
    lo = wid * rpw
    lanes = lax.iota(jnp.int32, LL)

    pltpu.sync_copy(zacc_h, acc)
    zi = jnp.zeros((LL,), jnp.int32)

    def zrow(i, c):
        slist[pl.ds(i * LL, LL)] = zi
        tlist[pl.ds(i * LL, LL)] = zi
        elist[pl.ds(i * LL, LL)] = zi
        return c

    lax.fori_loop(0, (ch + LL) // LL, zrow, jnp.int32(0))

    def chunk(cidx, carry):
        cbase = cidx * ch
        pltpu.sync_copy(tgt_h.at[pl.ds(cbase, ch)], tgtb)
        pltpu.sync_copy(src_h.at[pl.ds(cbase, ch)], srcb)

        def vec(v, cnt):
            t = tgtb[pl.ds(v * LL, LL)]
            msk = (t >= lo) & (t < lo + rpw)
            sv = srcb[pl.ds(v * LL, LL)]
            ev = _splat(cbase + v * LL) + lanes
            plsc.store_compressed(tlist.at[pl.ds(cnt, LL)], t - lo, mask=msk)
            plsc.store_compressed(slist.at[pl.ds(cnt, LL)], sv, mask=msk)
            plsc.store_compressed(elist.at[pl.ds(cnt, LL)], ev, mask=msk)
            pc = plsc.all_reduce_population_count(msk)
            return cnt + pc[0]

        cnt = lax.fori_loop(0, ch // LL, vec, jnp.int32(0))

        def group(g, carry):
            gbase = g * gsz
            gn = jnp.minimum(jnp.int32(gsz), cnt - gbase)
            pltpu.async_copy(nt_h.at[slist.at[pl.ds(gbase, gsz)]], ntb, sem).wait()
            pltpu.async_copy(ex_h.at[elist.at[pl.ds(gbase, gsz)]], exb, sem).wait()

            def edge(i, carry):
                tl16 = plsc.load_gather(tlist, [_splat(gbase + i)])
                for c in range(FT // LL):
                    if c % 2 == 0:
                        mult = plsc.load_gather(exb, [_splat(i), _splat(c // 2)])
                    cv = jnp.full((LL,), c * LL, jnp.int32) + lanes
                    v = plsc.load_gather(ntb, [_splat(i), cv])
                    plsc.addupdate_scatter(acc, [tl16, cv], v * mult)
                return carry

            return lax.fori_loop(0, gn, edge, carry)

        carry = lax.fori_loop(0, (cnt + gsz - 1) // gsz, group, carry)
        return carry

    lax.fori_loop(0, nch, chunk, jnp.int32(0))

    pltpu.sync_copy(den_h.at[pl.ds(lo, rpw)], d0b)
    pltpu.sync_copy(den_h.at[pl.ds(npad + lo, rpw)], d1b)

    gd = lax.GatherDimensionNumbers(offset_dims=(), collapsed_slice_dims=(0,),
                                    start_index_map=(0,))

    def fin(nl, carry):
        d0 = plsc.load_gather(d0b, [_splat(nl), lanes])
        d1 = plsc.load_gather(d1b, [_splat(nl), lanes])
        rec = jnp.float32(1.0) / (d0 + d1 + jnp.float32(1e-8))
        for c in range(FT // LL):
            if c % 2 == 0:
                dm = lax.gather(rec, jnp.full((LL, 1), c // 2, jnp.int32), gd,
                                (1,), mode=lax.GatherScatterMode.PROMISE_IN_BOUNDS)
            cv = jnp.full((LL,), c * LL, jnp.int32) + lanes
            a = plsc.load_gather(acc, [_splat(nl), cv])
            plsc.store_scatter(acc, [_splat(nl), cv], a * dm)
        return carry

    lax.fori_loop(0, rpw, fin, jnp.int32(0))
    pltpu.sync_copy(acc, out_h.at[pl.ds(lo, rpw)])


def _sc_aggregate(src, tgt, ex2d, nt, den2d, zacc, npad):
    rpw = npad // NW
    ch = 2000
    kfn = pl.kernel(
        _p3_body,
        out_type=jax.ShapeDtypeStruct((npad, FT), jnp.float32),
        mesh=_mesh(),
        compiler_params=pltpu.CompilerParams(
            needs_layout_passes=False, use_tc_tiling_on_sc=False),
        scratch_types=[
            pltpu.VMEM((rpw, FT), jnp.float32),
            pltpu.VMEM((ch,), jnp.int32),
            pltpu.VMEM((ch,), jnp.int32),
            pltpu.VMEM((ch + LL,), jnp.int32),
            pltpu.VMEM((ch + LL,), jnp.int32),
            pltpu.VMEM((ch + LL,), jnp.int32),
            pltpu.VMEM((64, FT), jnp.float32),
            pltpu.VMEM((64, LL), jnp.float32),
            pltpu.VMEM((rpw, LL), jnp.float32),
            pltpu.VMEM((rpw, LL), jnp.float32),
            pltpu.SemaphoreType.DMA,
        ],
    )
    return kfn(src, tgt, ex2d, nt, den2d, zacc)


@jax.jit
def kernel(x, edge_index, W, A):
    n = x.shape[0]
    rpw = -(-n // NW)
    npad = rpw * NW
    src = edge_index[0]
    tgt = edge_index[1]

    a4 = A.reshape(HH, 2, OF, HH)
    pad = jnp.zeros((FT, LL - HH), jnp.float32)
    a_s = jnp.concatenate([a4[:, 0].reshape(FT, HH), pad], axis=1)
    a_t = jnp.concatenate([a4[:, 1].reshape(FT, HH), pad], axis=1)

    nt, als, alt = _tc_project(x, W, a_s, a_t)

    att, mx = _sc_attmax(src, tgt, als, alt)

    zden = jnp.zeros((npad // NS, LL), jnp.float32)
    exf, den = _sc_softmax_denom(tgt, att, mx, zden, npad)

    zacc = jnp.zeros((rpw, FT), jnp.float32)
    outp = _sc_aggregate(src, tgt, exf, nt, den, zacc, npad)
    return outp[:n]


# speedup vs baseline: 10.6931x; 1.1589x over previous
import functools

import jax
import jax.numpy as jnp
from jax import lax
from jax.experimental import pallas as pl
from jax.experimental.pallas import tpu as pltpu
from jax.experimental.pallas import tpu_sc as plsc

NF = 128
HH = 8
OF = 32
FT = HH * OF
NC = 2
NS = 16
NW = NC * NS
LL = 16

_NEG = -3.4e38


def _mesh():
    return plsc.VectorSubcoreMesh(core_axis_name="c", subcore_axis_name="s",
                                  num_cores=NC, num_subcores=NS)


def _wid():
    return lax.axis_index("s") * NC + lax.axis_index("c")


def _splat(i):
    return jnp.full((LL,), i, jnp.int32)


def _tc_body(x_ref, w_ref, as_ref, at_ref, nt_ref, als_ref, alt_ref):
    ntv = jnp.dot(x_ref[...], w_ref[...], preferred_element_type=jnp.float32)
    nt_ref[...] = ntv
    als_ref[...] = jnp.dot(ntv, as_ref[...], preferred_element_type=jnp.float32)
    alt_ref[...] = jnp.dot(ntv, at_ref[...], preferred_element_type=jnp.float32)


def _tc_project(x, w, a_s, a_t):
    n = x.shape[0]
    blk = 2000
    grid = (n // blk,)
    return pl.pallas_call(
        _tc_body,
        grid=grid,
        in_specs=[
            pl.BlockSpec((blk, NF), lambda i: (i, 0)),
            pl.BlockSpec((NF, FT), lambda i: (0, 0)),
            pl.BlockSpec((FT, LL), lambda i: (0, 0)),
            pl.BlockSpec((FT, LL), lambda i: (0, 0)),
        ],
        out_specs=[
            pl.BlockSpec((blk, FT), lambda i: (i, 0)),
            pl.BlockSpec((blk, LL), lambda i: (i, 0)),
            pl.BlockSpec((blk, LL), lambda i: (i, 0)),
        ],
        out_shape=[
            jax.ShapeDtypeStruct((n, FT), jnp.float32),
            jax.ShapeDtypeStruct((n, LL), jnp.float32),
            jax.ShapeDtypeStruct((n, LL), jnp.float32),
        ],
    )(x, w, a_s, a_t)


def _p1_body(src_h, tgt_h, als_h, alt_h, att_h, max_h,
             sidx, tidx, g1, g2, g3, sem):
    e = src_h.shape[0]
    epw = e // NW
    bsz = 400
    nb = epw // bsz
    wid = _wid()
    base0 = wid * epw
    lanes = lax.iota(jnp.int32, LL)
    lo_mask = lanes < HH

    def batch(b, mx):
        base = base0 + b * bsz
        c1 = pltpu.async_copy(src_h.at[pl.ds(base, bsz)], sidx, sem)
        c2 = pltpu.async_copy(tgt_h.at[pl.ds(base, bsz)], tidx, sem)
        c1.wait()
        c2.wait()
        c3 = pltpu.async_copy(als_h.at[sidx], g1, sem)
        c4 = pltpu.async_copy(alt_h.at[tidx], g2, sem)
        c3.wait()
        c4.wait()

        def row(i, mx):
            a = plsc.load_gather(g1, [_splat(i), lanes])
            bt = plsc.load_gather(g2, [_splat(i), lanes])
            v = a + bt
            g3[pl.ds(i * LL, LL)] = v
            sel = jnp.where(lo_mask, v, _NEG)
            return jnp.maximum(mx, sel)

        mx = lax.fori_loop(0, bsz, row, mx)
        pltpu.sync_copy(g3, att_h.at[pl.ds(base * LL, bsz * LL)])
        return mx

    mx = lax.fori_loop(0, nb, batch, jnp.full((LL,), _NEG, jnp.float32))
    g3[pl.ds(0, LL)] = mx
    pltpu.sync_copy(g3.at[pl.ds(0, LL)], max_h.at[pl.ds(wid * LL, LL)])


def _sc_attmax(src, tgt, als, alt):
    e = src.shape[0]
    bsz = 400
    kfn = pl.kernel(
        _p1_body,
        out_type=[
            jax.ShapeDtypeStruct((e * LL,), jnp.float32),
            jax.ShapeDtypeStruct((NW * LL,), jnp.float32),
        ],
        mesh=_mesh(),
        compiler_params=pltpu.CompilerParams(
            needs_layout_passes=False, use_tc_tiling_on_sc=False),
        scratch_types=[
            pltpu.VMEM((bsz,), jnp.int32),
            pltpu.VMEM((bsz,), jnp.int32),
            pltpu.VMEM((bsz, LL), jnp.float32),
            pltpu.VMEM((bsz, LL), jnp.float32),
            pltpu.VMEM((bsz * LL,), jnp.float32),
            pltpu.SemaphoreType.DMA,
        ],
    )
    return kfn(src, tgt, als, alt)


def _p2_body(tgt_h, att_h, max_h, zden_h, ex_h, den_h,
             tl, ab, eb, mb, den_sh, sem):
    e = tgt_h.shape[0]
    npad = den_sh.shape[0]
    drs = npad // NS
    epw = e // NW
    bsz = 400
    nb = epw // bsz
    wid = _wid()
    cid = lax.axis_index("c")
    sid = lax.axis_index("s")
    base0 = wid * epw
    lanes = lax.iota(jnp.int32, LL)
    lo_mask = lanes < HH

    pltpu.sync_copy(max_h, mb)

    def mred(i, mx):
        return jnp.maximum(mx, mb[pl.ds(i * LL, LL)])

    m16 = lax.fori_loop(0, NW, mred, jnp.full((LL,), _NEG, jnp.float32))
    m = jnp.max(m16)
    msp = jnp.full((LL,), m, jnp.float32)

    pltpu.sync_copy(zden_h, den_sh.at[pl.ds(sid * drs, drs)])
    plsc.subcore_barrier()

    def batch(b, carry):
        base = base0 + b * bsz
        c1 = pltpu.async_copy(tgt_h.at[pl.ds(base, bsz)], tl, sem)
        c2 = pltpu.async_copy(att_h.at[pl.ds(base * LL, bsz * LL)], ab, sem)
        c1.wait()
        c2.wait()

        def row(i, carry):
            v = ab[pl.ds(i * LL, LL)]
            d = v - msp
            w = jnp.where(d >= 0.0, d, d * jnp.float32(0.01))
            ev = jnp.exp(w)
            ev = jnp.where(lo_mask, ev, jnp.float32(0.0))
            plsc.store_scatter(eb, [_splat(i), lanes], ev)
            return carry

        carry = lax.fori_loop(0, bsz, row, carry)
        pltpu.sync_copy(eb, ex_h.at[pl.ds(base, bsz)])
        pltpu.sync_copy(eb, den_sh.at[tl], add=True)
        return carry

    lax.fori_loop(0, nb, batch, jnp.int32(0))
    plsc.subcore_barrier()
    pltpu.sync_copy(den_sh.at[pl.ds(sid * drs, drs)],
                    den_h.at[pl.ds(cid * npad + sid * drs, drs)])


def _sc_softmax_denom(tgt, att, mx, zden, npad):
    e = tgt.shape[0]
    bsz = 400
    kfn = pl.kernel(
        _p2_body,
        out_type=[
            jax.ShapeDtypeStruct((e, LL), jnp.float32),
            jax.ShapeDtypeStruct((NC * npad, LL), jnp.float32),
        ],
        mesh=_mesh(),
        compiler_params=pltpu.CompilerParams(
            needs_layout_passes=False, use_tc_tiling_on_sc=False),
        scratch_types=[
            pltpu.VMEM((bsz,), jnp.int32),
            pltpu.VMEM((bsz * LL,), jnp.float32),
            pltpu.VMEM((bsz, LL), jnp.float32),
            pltpu.VMEM((NW * LL,), jnp.float32),
            pltpu.VMEM_SHARED((npad, LL), jnp.float32),
            pltpu.SemaphoreType.DMA,
        ],
    )
    return kfn(tgt, att, mx, zden)


def _p3_body(src_h, tgt_h, ex_h, nt_h, den_h, zacc_h, out_h,
             acc, tgtb, srcb, slist, tlist, elist, ntb, exb,
             d0b, d1b, sem):
    e = src_h.shape[0]
    npad = out_h.shape[0]
    rpw = npad // NW
    ch = 2000
    nch = e // ch
    gsz = 96
    wid = _wid()
    lo = wid * rpw
    lanes = lax.iota(jnp.int32, LL)

    pltpu.sync_copy(zacc_h, acc)
    zi = jnp.zeros((LL,), jnp.int32)

    def zrow(i, c):
        slist[pl.ds(i * LL, LL)] = zi
        tlist[pl.ds(i * LL, LL)] = zi
        elist[pl.ds(i * LL, LL)] = zi
        return c

    lax.fori_loop(0, (ch + LL) // LL, zrow, jnp.int32(0))

    def chunk(cidx, carry):
        cbase = cidx * ch
        c1 = pltpu.async_copy(tgt_h.at[pl.ds(cbase, ch)], tgtb, sem)
        c2 = pltpu.async_copy(src_h.at[pl.ds(cbase, ch)], srcb, sem)
        c1.wait()
        c2.wait()

        def vec(v, cnt):
            t = tgtb[pl.ds(v * LL, LL)]
            msk = (t >= lo) & (t < lo + rpw)
            sv = srcb[pl.ds(v * LL, LL)]
            ev = _splat(cbase + v * LL) + lanes
            plsc.store_compressed(tlist.at[pl.ds(cnt, LL)], t - lo, mask=msk)
            plsc.store_compressed(slist.at[pl.ds(cnt, LL)], sv, mask=msk)
            plsc.store_compressed(elist.at[pl.ds(cnt, LL)], ev, mask=msk)
            pc = plsc.all_reduce_population_count(msk)
            return cnt + pc[0]

        cnt = lax.fori_loop(0, ch // LL, vec, jnp.int32(0))

        def group(g, carry):
            gbase = g * gsz
            gn = jnp.minimum(jnp.int32(gsz), cnt - gbase)
            g1 = pltpu.async_copy(nt_h.at[slist.at[pl.ds(gbase, gsz)]], ntb, sem)
            g2 = pltpu.async_copy(ex_h.at[elist.at[pl.ds(gbase, gsz)]], exb, sem)
            g1.wait()
            g2.wait()

            def edge(i, carry):
                tl16 = plsc.load_gather(tlist, [_splat(gbase + i)])
                for c in range(FT // LL):
                    if c % 2 == 0:
                        mult = plsc.load_gather(exb, [_splat(i), _splat(c // 2)])
                    cv = jnp.full((LL,), c * LL, jnp.int32) + lanes
                    v = plsc.load_gather(ntb, [_splat(i), cv])
                    plsc.addupdate_scatter(acc, [tl16, cv], v * mult)
                return carry

            return lax.fori_loop(0, gn, edge, carry)

        carry = lax.fori_loop(0, (cnt + gsz - 1) // gsz, group, carry)
        return carry

    lax.fori_loop(0, nch, chunk, jnp.int32(0))

    pltpu.sync_copy(den_h.at[pl.ds(lo, rpw)], d0b)
    pltpu.sync_copy(den_h.at[pl.ds(npad + lo, rpw)], d1b)

    gd = lax.GatherDimensionNumbers(offset_dims=(), collapsed_slice_dims=(0,),
                                    start_index_map=(0,))

    def fin(nl, carry):
        d0 = plsc.load_gather(d0b, [_splat(nl), lanes])
        d1 = plsc.load_gather(d1b, [_splat(nl), lanes])
        rec = jnp.float32(1.0) / (d0 + d1 + jnp.float32(1e-8))
        for c in range(FT // LL):
            if c % 2 == 0:
                dm = lax.gather(rec, jnp.full((LL, 1), c // 2, jnp.int32), gd,
                                (1,), mode=lax.GatherScatterMode.PROMISE_IN_BOUNDS)
            cv = jnp.full((LL,), c * LL, jnp.int32) + lanes
            a = plsc.load_gather(acc, [_splat(nl), cv])
            plsc.store_scatter(acc, [_splat(nl), cv], a * dm)
        return carry

    lax.fori_loop(0, rpw, fin, jnp.int32(0))
    pltpu.sync_copy(acc, out_h.at[pl.ds(lo, rpw)])


def _sc_aggregate(src, tgt, ex2d, nt, den2d, zacc, npad):
    rpw = npad // NW
    ch = 2000
    kfn = pl.kernel(
        _p3_body,
        out_type=jax.ShapeDtypeStruct((npad, FT), jnp.float32),
        mesh=_mesh(),
        compiler_params=pltpu.CompilerParams(
            needs_layout_passes=False, use_tc_tiling_on_sc=False),
        scratch_types=[
            pltpu.VMEM((rpw, FT), jnp.float32),
            pltpu.VMEM((ch,), jnp.int32),
            pltpu.VMEM((ch,), jnp.int32),
            pltpu.VMEM((ch + LL,), jnp.int32),
            pltpu.VMEM((ch + LL,), jnp.int32),
            pltpu.VMEM((ch + LL,), jnp.int32),
            pltpu.VMEM((96, FT), jnp.float32),
            pltpu.VMEM((96, LL), jnp.float32),
            pltpu.VMEM((rpw, LL), jnp.float32),
            pltpu.VMEM((rpw, LL), jnp.float32),
            pltpu.SemaphoreType.DMA,
        ],
    )
    return kfn(src, tgt, ex2d, nt, den2d, zacc)


@jax.jit
def kernel(x, edge_index, W, A):
    n = x.shape[0]
    rpw = -(-n // NW)
    npad = rpw * NW
    src = edge_index[0]
    tgt = edge_index[1]

    a4 = A.reshape(HH, 2, OF, HH)
    pad = jnp.zeros((FT, LL - HH), jnp.float32)
    a_s = jnp.concatenate([a4[:, 0].reshape(FT, HH), pad], axis=1)
    a_t = jnp.concatenate([a4[:, 1].reshape(FT, HH), pad], axis=1)

    nt, als, alt = _tc_project(x, W, a_s, a_t)

    att, mx = _sc_attmax(src, tgt, als, alt)

    zden = jnp.zeros((npad // NS, LL), jnp.float32)
    exf, den = _sc_softmax_denom(tgt, att, mx, zden, npad)

    zacc = jnp.zeros((rpw, FT), jnp.float32)
    outp = _sc_aggregate(src, tgt, exf, nt, den, zacc, npad)
    return outp[:n]
